# baseline (speedup 1.0000x reference)
import functools

import jax
import jax.numpy as jnp
from jax import lax
from jax.experimental import pallas as pl
from jax.experimental.pallas import tpu as pltpu
from jax.experimental.pallas import tpu_sc as plsc

_L = 16
_W = 128
_NS = 4


def _body(nc, bpw, d,
          ueT, ieT, ulT, ilT, u, i,
          zu_o, su_o, zi_o, si_o,
          u_v, i_v, we_v, wl_v, zu_v, su_v, zi_v, si_v,
          sems_e, sems_l, sem_o):
    wid = lax.axis_index("s") * nc + lax.axis_index("c")
    base = wid * bpw
    ng = bpw // _L

    pltpu.sync_copy(u.at[pl.ds(base, bpw)], u_v)
    pltpu.sync_copy(i.at[pl.ds(base, bpw)], i_v)

    clo = lax.iota(jnp.int32, _L)
    chi = clo + _L
    zeros = clo - clo

    def win(r):
        return pl.ds(pl.multiple_of((r >> 7) * _W, _W), _W)

    def fire(emb, logs, r, slot):
        w = win(r)
        pltpu.async_copy(emb.at[:, w], we_v.at[slot], sems_e[slot])
        pltpu.async_copy(logs.at[:, w], wl_v.at[slot], sems_l[slot])

    def run_pair(idx_ref, emb, logs, z_v, s_v):
        g0 = idx_ref[pl.ds(0, _L)]
        for l in range(_NS):
            fire(emb, logs, g0[l], l)

        def step(t, carry):
            gvec = idx_ref[pl.ds(t * _L, _L)]
            tn = jnp.minimum(t + 1, ng - 1)
            gnext = idx_ref[pl.ds(tn * _L, _L)]
            for l in range(_L):
                slot = l & (_NS - 1)
                b = t * _L + l
                pltpu.make_async_copy(
                    emb.at[:, pl.ds(0, _W)], we_v.at[slot], sems_e[slot]
                ).wait()
                pltpu.make_async_copy(
                    emb.at[:, pl.ds(0, _W)], wl_v.at[slot], sems_l[slot]
                ).wait()
                ovec = zeros + (gvec[l] & (_W - 1))
                bvec = zeros + b
                for cvec in (clo, chi):
                    ve = plsc.load_gather(we_v.at[slot], [cvec, ovec])
                    plsc.store_scatter(z_v, [cvec, bvec], ve)
                    vl = plsc.load_gather(wl_v.at[slot], [cvec, ovec])
                    plsc.store_scatter(s_v, [cvec, bvec], jnp.exp(vl))
                rn = gvec[l + _NS] if l < _L - _NS else gnext[l - (_L - _NS)]

                @pl.when(b + _NS < bpw)
                def _():
                    fire(emb, logs, rn, slot)
            return carry

        lax.fori_loop(0, ng, step, 0)

    run_pair(u_v, ueT, ulT, zu_v, su_v)
    run_pair(i_v, ieT, ilT, zi_v, si_v)

    ob = pl.ds(base, bpw)
    out_zu = pltpu.async_copy(zu_v, zu_o.at[:, ob], sem_o)
    out_su = pltpu.async_copy(su_v, su_o.at[:, ob], sem_o)
    out_zi = pltpu.async_copy(zi_v, zi_o.at[:, ob], sem_o)
    out_si = pltpu.async_copy(si_v, si_o.at[:, ob], sem_o)
    out_zu.wait()
    out_su.wait()
    out_zi.wait()
    out_si.wait()


def kernel(user_emb, item_emb, user_logsigma, item_logsigma, u, i):
    info = plsc.get_sparse_core_info()
    nc, ns = info.num_cores, info.num_subcores
    nw = nc * ns
    b = u.shape[0]
    nrows, d = user_emb.shape
    bpw = b // nw

    out = jax.ShapeDtypeStruct((d, b), jnp.float32)
    mesh = plsc.VectorSubcoreMesh(core_axis_name="c", subcore_axis_name="s")
    f = pl.kernel(
        functools.partial(_body, nc, bpw, d),
        out_type=(out, out, out, out),
        mesh=mesh,
        compiler_params=pltpu.CompilerParams(needs_layout_passes=False),
        scratch_types=[
            pltpu.VMEM((bpw,), jnp.int32),
            pltpu.VMEM((bpw,), jnp.int32),
            pltpu.VMEM((_NS, d, _W), jnp.float32),
            pltpu.VMEM((_NS, d, _W), jnp.float32),
            pltpu.VMEM((d, bpw), jnp.float32),
            pltpu.VMEM((d, bpw), jnp.float32),
            pltpu.VMEM((d, bpw), jnp.float32),
            pltpu.VMEM((d, bpw), jnp.float32),
            [pltpu.SemaphoreType.DMA] * _NS,
            [pltpu.SemaphoreType.DMA] * _NS,
            pltpu.SemaphoreType.DMA,
        ],
    )
    zuT, suT, ziT, siT = f(user_emb.T, item_emb.T, user_logsigma.T,
                           item_logsigma.T, u.astype(jnp.int32),
                           i.astype(jnp.int32))
    return (zuT.T, suT.T, ziT.T, siT.T)


# speedup vs baseline: 1.0898x; 1.0898x over previous
import functools

import jax
import jax.numpy as jnp
from jax import lax
from jax.experimental import pallas as pl
from jax.experimental.pallas import tpu as pltpu
from jax.experimental.pallas import tpu_sc as plsc

_L = 16
_W = 128
_NS = 4


def _body(nc, bpw, d,
          ueT, ieT, ulT, ilT, u, i,
          zu_o, su_o, zi_o, si_o,
          u_v, i_v, weu_v, wlu_v, wei_v, wli_v,
          gzu_v, gsu_v, gzi_v, gsi_v,
          sems_u, sems_i, sems_o):
    wid = lax.axis_index("s") * nc + lax.axis_index("c")
    base = wid * bpw
    ng = bpw // _L

    pltpu.sync_copy(u.at[pl.ds(base, bpw)], u_v)
    pltpu.sync_copy(i.at[pl.ds(base, bpw)], i_v)

    clo = lax.iota(jnp.int32, _L)
    chi = clo + _L
    zeros = clo - clo

    def win(r):
        return pl.ds(pl.multiple_of((r >> 7) * _W, _W), _W)

    def fire_u(r, slot):
        w = win(r)
        pltpu.async_copy(ueT.at[:, w], weu_v.at[slot], sems_u[slot])
        pltpu.async_copy(ulT.at[:, w], wlu_v.at[slot], sems_u[slot])

    def fire_i(r, slot):
        w = win(r)
        pltpu.async_copy(ieT.at[:, w], wei_v.at[slot], sems_i[slot])
        pltpu.async_copy(ilT.at[:, w], wli_v.at[slot], sems_i[slot])

    def drain2(wa, wb, sem):
        pltpu.make_async_copy(ueT.at[:, pl.ds(0, _W)], wa, sem).wait()
        pltpu.make_async_copy(ueT.at[:, pl.ds(0, _W)], wb, sem).wait()

    def owait(p):
        for g in (gzu_v, gsu_v, gzi_v, gsi_v):
            pltpu.make_async_copy(ueT.at[:, pl.ds(0, _W)], g.at[p], sems_o[p]
                                  ).wait()

    gu0 = u_v[pl.ds(0, _L)]
    gi0 = i_v[pl.ds(0, _L)]
    for l in range(_NS):
        fire_u(gu0[l], l)
        fire_i(gi0[l], l)

    def step(t, carry):
        p = (t >> 3) & 1
        gu = u_v[pl.ds(t * _L, _L)]
        gi = i_v[pl.ds(t * _L, _L)]
        tn = jnp.minimum(t + 1, ng - 1)
        gun = u_v[pl.ds(tn * _L, _L)]
        gin = i_v[pl.ds(tn * _L, _L)]

        blk_start = (t & 7) == 0

        @pl.when(jnp.logical_and(blk_start, jnp.logical_and(t >= 16, p == 0)))
        def _():
            owait(0)

        @pl.when(jnp.logical_and(blk_start, jnp.logical_and(t >= 16, p == 1)))
        def _():
            owait(1)

        for l in range(_L):
            slot = l % _NS
            lvec = zeros + ((t & 7) * _L + l)
            drain2(weu_v.at[slot], wlu_v.at[slot], sems_u[slot])
            ovec = zeros + (gu[l] & (_W - 1))
            for cvec in (clo, chi):
                ve = plsc.load_gather(weu_v.at[slot], [cvec, ovec])
                plsc.store_scatter(gzu_v.at[p], [cvec, lvec], ve)
                vl = plsc.load_gather(wlu_v.at[slot], [cvec, ovec])
                plsc.store_scatter(gsu_v.at[p], [cvec, lvec], jnp.exp(vl))
            drain2(wei_v.at[slot], wli_v.at[slot], sems_i[slot])
            qvec = zeros + (gi[l] & (_W - 1))
            for cvec in (clo, chi):
                ve = plsc.load_gather(wei_v.at[slot], [cvec, qvec])
                plsc.store_scatter(gzi_v.at[p], [cvec, lvec], ve)
                vl = plsc.load_gather(wli_v.at[slot], [cvec, qvec])
                plsc.store_scatter(gsi_v.at[p], [cvec, lvec], jnp.exp(vl))
            b = t * _L + l
            rnu = gu[l + _NS] if l < _L - _NS else gun[l - (_L - _NS)]
            rni = gi[l + _NS] if l < _L - _NS else gin[l - (_L - _NS)]

            @pl.when(b + _NS < bpw)
            def _():
                fire_u(rnu, slot)
                fire_i(rni, slot)

        ob = pl.ds(base + (t >> 3) * _W, _W)
        blk_end = (t & 7) == 7

        @pl.when(jnp.logical_and(blk_end, p == 0))
        def _():
            pltpu.async_copy(gzu_v.at[0], zu_o.at[:, ob], sems_o[0])
            pltpu.async_copy(gsu_v.at[0], su_o.at[:, ob], sems_o[0])
            pltpu.async_copy(gzi_v.at[0], zi_o.at[:, ob], sems_o[0])
            pltpu.async_copy(gsi_v.at[0], si_o.at[:, ob], sems_o[0])

        @pl.when(jnp.logical_and(blk_end, p == 1))
        def _():
            pltpu.async_copy(gzu_v.at[1], zu_o.at[:, ob], sems_o[1])
            pltpu.async_copy(gsu_v.at[1], su_o.at[:, ob], sems_o[1])
            pltpu.async_copy(gzi_v.at[1], zi_o.at[:, ob], sems_o[1])
            pltpu.async_copy(gsi_v.at[1], si_o.at[:, ob], sems_o[1])
        return carry

    lax.fori_loop(0, ng, step, 0)
    owait(0)
    owait(1)


def kernel(user_emb, item_emb, user_logsigma, item_logsigma, u, i):
    info = plsc.get_sparse_core_info()
    nc, ns = info.num_cores, info.num_subcores
    nw = nc * ns
    b = u.shape[0]
    nrows, d = user_emb.shape
    bpw = b // nw

    out = jax.ShapeDtypeStruct((d, b), jnp.float32)
    mesh = plsc.VectorSubcoreMesh(core_axis_name="c", subcore_axis_name="s")
    f = pl.kernel(
        functools.partial(_body, nc, bpw, d),
        out_type=(out, out, out, out),
        mesh=mesh,
        compiler_params=pltpu.CompilerParams(needs_layout_passes=False),
        scratch_types=[
            pltpu.VMEM((bpw,), jnp.int32),
            pltpu.VMEM((bpw,), jnp.int32),
            pltpu.VMEM((_NS, d, _W), jnp.float32),
            pltpu.VMEM((_NS, d, _W), jnp.float32),
            pltpu.VMEM((_NS, d, _W), jnp.float32),
            pltpu.VMEM((_NS, d, _W), jnp.float32),
            pltpu.VMEM((2, d, _W), jnp.float32),
            pltpu.VMEM((2, d, _W), jnp.float32),
            pltpu.VMEM((2, d, _W), jnp.float32),
            pltpu.VMEM((2, d, _W), jnp.float32),
            [pltpu.SemaphoreType.DMA] * _NS,
            [pltpu.SemaphoreType.DMA] * _NS,
            [pltpu.SemaphoreType.DMA] * 2,
        ],
    )
    zuT, suT, ziT, siT = f(user_emb.T, item_emb.T, user_logsigma.T,
                           item_logsigma.T, u.astype(jnp.int32),
                           i.astype(jnp.int32))
    return (zuT.T, suT.T, ziT.T, siT.T)
